# baseline (speedup 1.0000x reference)
import functools
from math import comb

import jax
import jax.numpy as jnp
from jax import lax
from jax.experimental import pallas as pl
from jax.experimental.pallas import tpu as pltpu
from jax.experimental.pallas import tpu_sc as plsc

N = 10000
E = 320000
D = 128
K = 10

NC = 2
NS = 16
NW = NC * NS
CL = 128
BLKE = 2048
CPBLK = BLKE // CL
E_PAD = 327680
RPT = 316
N_PAD = NW * RPT
TROWS = RPT + 1
ACCW = TROWS * D

_mesh = plsc.VectorSubcoreMesh(core_axis_name="c", subcore_axis_name="s")


@functools.partial(
    pl.kernel,
    mesh=_mesh,
    out_type=jax.ShapeDtypeStruct((N_PAD * D,), jnp.float32),
    scratch_types=[
        pltpu.VMEM((NW * 16,), jnp.int32),
        pltpu.VMEM((BLKE,), jnp.int32),
        pltpu.VMEM((BLKE,), jnp.int32),
        pltpu.VMEM((CL, D), jnp.float32),
        pltpu.VMEM((CL, D), jnp.float32),
        pltpu.VMEM((ACCW,), jnp.float32),
        pltpu.SemaphoreType.DMA,
        pltpu.SemaphoreType.DMA,
    ],
)
def _sc_spmm(g_hbm, col_hbm, row_hbm, bnd_hbm, out_hbm,
             bndv, colblk, rowblk, gbuf0, gbuf1, accf, gsem0, gsem1):
    c = lax.axis_index("c")
    s = lax.axis_index("s")
    w = s * NC + c
    gbufs = (gbuf0, gbuf1)
    gsems = (gsem0, gsem1)

    pltpu.sync_copy(bnd_hbm, bndv)

    z16 = jnp.zeros((16,), jnp.float32)

    def zero_body(i, carry):
        accf[pl.ds(i * 16, 16)] = z16
        return carry

    lax.fori_loop(0, ACCW // 16, zero_body, 0)

    bv = bndv[pl.ds(w * 16, 16)]
    e0 = bv[0]
    e1 = bv[1]
    estart = (e0 // BLKE) * BLKE
    nblk = ((e1 + BLKE - 1) // BLKE) * BLKE // BLKE - e0 // BLKE

    rbase = w * RPT

    def blk_body(bi, carry):
        eb = estart + bi * BLKE
        pltpu.sync_copy(col_hbm.at[pl.ds(eb, BLKE)], colblk)
        pltpu.sync_copy(row_hbm.at[pl.ds(eb, BLKE)], rowblk)
        for b in range(2):
            pltpu.make_async_copy(
                g_hbm.at[colblk.at[pl.ds(b * CL, CL)]], gbufs[b], gsems[b]
            ).start()
        for ci in range(CPBLK):
            b = ci % 2
            pltpu.make_async_copy(
                g_hbm.at[colblk.at[pl.ds(ci * CL, CL)]], gbufs[b], gsems[b]
            ).wait()

            def grp_body(gi, carry2, _ci=ci, _b=b):
                rvec = rowblk[pl.ds(_ci * CL + gi * 16, 16)]
                for j in range(16):
                    lr = rvec[j] - rbase
                    base = jnp.where((lr >= 0) & (lr < RPT), lr, RPT) * D
                    er = gi * 16 + j
                    for fg in range(8):
                        x = gbufs[_b][er, pl.ds(fg * 16, 16)]
                        plsc.addupdate(accf.at[pl.ds(base + fg * 16, 16)], x)
                return carry2

            lax.fori_loop(0, CL // 16, grp_body, 0)
            if ci + 2 < CPBLK:
                pltpu.make_async_copy(
                    g_hbm.at[colblk.at[pl.ds((ci + 2) * CL, CL)]],
                    gbufs[b], gsems[b]).start()
        return carry

    lax.fori_loop(0, nblk, blk_body, 0)
    pltpu.sync_copy(accf.at[pl.ds(0, RPT * D)],
                    out_hbm.at[pl.ds(w * RPT * D, RPT * D)])


@functools.partial(
    pl.kernel,
    mesh=_mesh,
    out_type=jax.ShapeDtypeStruct((N_PAD * D,), jnp.float32),
    scratch_types=[
        pltpu.VMEM((NW * 16,), jnp.int32),
        pltpu.VMEM((BLKE,), jnp.int32),
        pltpu.VMEM((ACCW,), jnp.float32),
    ],
)
def _sc_degree(row_hbm, bnd_hbm, out_hbm, bndv, rowblk, accf):
    c = lax.axis_index("c")
    s = lax.axis_index("s")
    w = s * NC + c

    pltpu.sync_copy(bnd_hbm, bndv)

    z16 = jnp.zeros((16,), jnp.float32)

    def zero_body(i, carry):
        accf[pl.ds(i * 16, 16)] = z16
        return carry

    lax.fori_loop(0, ACCW // 16, zero_body, 0)

    bv = bndv[pl.ds(w * 16, 16)]
    e0 = bv[0]
    e1 = bv[1]
    estart = (e0 // BLKE) * BLKE
    nblk = ((e1 + BLKE - 1) // BLKE) * BLKE // BLKE - e0 // BLKE

    iota = lax.iota(jnp.int32, 16)
    onehot = jnp.where(iota == 0, 1.0, 0.0).astype(jnp.float32)
    rbase = w * RPT

    def blk_body(bi, carry):
        eb = estart + bi * BLKE
        pltpu.sync_copy(row_hbm.at[pl.ds(eb, BLKE)], rowblk)

        def grp_body(gi, carry2):
            rvec = rowblk[pl.ds(gi * 16, 16)]
            for j in range(16):
                lr = rvec[j] - rbase
                base = jnp.where((lr >= 0) & (lr < RPT), lr, RPT) * D
                plsc.addupdate(accf.at[pl.ds(base, 16)], onehot)
            return carry2

        lax.fori_loop(0, BLKE // 16, grp_body, 0)
        return carry

    lax.fori_loop(0, nblk, blk_body, 0)
    pltpu.sync_copy(accf.at[pl.ds(0, RPT * D)],
                    out_hbm.at[pl.ds(w * RPT * D, RPT * D)])


BR = 400
GRID = N // BR


def _tc_pre_body(x_ref, dp_ref, g0_ref, d2_ref, sqd_ref):
    deg = dp_ref[...] + 1.0
    dinv = lax.rsqrt(deg)
    g0_ref[...] = x_ref[...] * dinv
    d2_ref[...] = 1.0 / deg
    sqd_ref[...] = deg * dinv


def _tc_precompute(x, dp):
    return pl.pallas_call(
        _tc_pre_body,
        grid=(GRID,),
        in_specs=[
            pl.BlockSpec((BR, D), lambda i: (i, 0)),
            pl.BlockSpec((BR, 1), lambda i: (i, 0)),
        ],
        out_specs=[
            pl.BlockSpec((BR, D), lambda i: (i, 0)),
            pl.BlockSpec((BR, 1), lambda i: (i, 0)),
            pl.BlockSpec((BR, 1), lambda i: (i, 0)),
        ],
        out_shape=[
            jax.ShapeDtypeStruct((N, D), jnp.float32),
            jax.ShapeDtypeStruct((N, 1), jnp.float32),
            jax.ShapeDtypeStruct((N, 1), jnp.float32),
        ],
    )(x, dp)


def _tc_update_body(g_ref, s_ref, d2_ref, out_ref):
    g = g_ref[...]
    stot = s_ref[...] + g
    out_ref[...] = 0.5 * g - 0.5 * d2_ref[...] * stot


def _tc_update(g, sarr, d2):
    return pl.pallas_call(
        _tc_update_body,
        grid=(GRID,),
        in_specs=[
            pl.BlockSpec((BR, D), lambda i: (i, 0)),
            pl.BlockSpec((BR, D), lambda i: (i, 0)),
            pl.BlockSpec((BR, 1), lambda i: (i, 0)),
        ],
        out_specs=pl.BlockSpec((BR, D), lambda i: (i, 0)),
        out_shape=jax.ShapeDtypeStruct((N, D), jnp.float32),
    )(g, sarr, d2)


CMAT = [[0.0] * (K + 1) for _ in range(K + 1)]
for k in range(K + 1):
    for j in range(K - k + 1):
        CMAT[k][k + j] = float(((-1) ** j) * comb(K, k) * comb(K - k, j))


def _tc_combine_body(sqd_ref, *refs):
    g_refs = refs[:K + 1]
    out_ref = refs[K + 1]
    sq = sqd_ref[...]
    gs = [r[...] for r in g_refs]
    for k in range(K + 1):
        acc = None
        for m in range(k, K + 1):
            term = CMAT[k][m] * gs[m]
            acc = term if acc is None else acc + term
        out_ref[k, :, :] = acc * sq


def _tc_combine(sqd, gs):
    in_specs = [pl.BlockSpec((BR, 1), lambda i: (i, 0))]
    in_specs += [pl.BlockSpec((BR, D), lambda i: (i, 0)) for _ in range(K + 1)]
    return pl.pallas_call(
        _tc_combine_body,
        grid=(GRID,),
        in_specs=in_specs,
        out_specs=pl.BlockSpec((K + 1, BR, D), lambda i: (0, i, 0)),
        out_shape=jax.ShapeDtypeStruct((K + 1, N, D), jnp.float32),
    )(sqd, *gs)


@jax.jit
def kernel(x, edge_index):
    row = edge_index[0].astype(jnp.int32)
    col = edge_index[1].astype(jnp.int32)
    pad = E_PAD - E
    row_p = jnp.concatenate([row, jnp.full((pad,), N, jnp.int32)])
    col_p = jnp.concatenate([col, jnp.zeros((pad,), jnp.int32)])
    perm = jnp.argsort(row_p)
    rs = row_p[perm]
    cs = col_p[perm]
    edges_bounds = jnp.searchsorted(
        rs, jnp.arange(NW + 1, dtype=jnp.int32) * RPT).astype(jnp.int32)
    bnd = (jnp.zeros((NW, 16), jnp.int32)
           .at[:, 0].set(edges_bounds[:NW])
           .at[:, 1].set(edges_bounds[1:]).reshape(-1))

    dflat = _sc_degree(rs, bnd)
    dp = dflat.reshape(N_PAD, D)[:N, 0:1]
    g0, d2, sqd = _tc_precompute(x, dp)

    gs = [g0]
    g = g0
    for _ in range(K):
        sflat = _sc_spmm(g, cs, rs, bnd)
        g = _tc_update(g, sflat.reshape(N_PAD, D)[:N], d2)
        gs.append(g)

    return _tc_combine(sqd, gs)


# speedup vs baseline: 1.2804x; 1.2804x over previous
import functools
from math import comb

import jax
import jax.numpy as jnp
from jax import lax
from jax.experimental import pallas as pl
from jax.experimental.pallas import tpu as pltpu
from jax.experimental.pallas import tpu_sc as plsc

N = 10000
E = 320000
D = 128
K = 10

NC = 2
NS = 16
NW = NC * NS
CL = 128
TCH = 80
E_PAD = TCH * NW * CL
N_PAD = 10112
ZPT = N_PAD // NS
NBUF = 2
CPB = 16
NBLK = TCH // CPB

_mesh = plsc.VectorSubcoreMesh(core_axis_name="c", subcore_axis_name="s")


@functools.partial(
    pl.kernel,
    mesh=_mesh,
    out_type=jax.ShapeDtypeStruct((NC * N_PAD, D), jnp.float32),
    scratch_types=[
        pltpu.VMEM((CPB, CL), jnp.int32),
        pltpu.VMEM((CPB, CL), jnp.int32),
    ] + [pltpu.VMEM((CL, D), jnp.float32)] * NBUF + [
        pltpu.VMEM_SHARED((N_PAD, D), jnp.float32),
    ] + [pltpu.SemaphoreType.DMA] * (2 * NBUF),
)
def _sc_spmm(g_hbm, col_hbm, row_hbm, zeros_hbm, out_hbm,
             colv, rowv, *rest):
    bufs = rest[:NBUF]
    acc = rest[NBUF]
    sems = rest[NBUF + 1:NBUF + 1 + NBUF]
    ssems = rest[NBUF + 1 + NBUF:]
    c = lax.axis_index("c")
    s = lax.axis_index("s")
    wid = s * NC + c
    pltpu.sync_copy(zeros_hbm.at[pl.ds(s * ZPT, ZPT)], acc.at[pl.ds(s * ZPT, ZPT)])
    plsc.subcore_barrier()

    def outer(bi, carry):
        base = wid * TCH + bi * CPB
        pltpu.sync_copy(col_hbm.at[pl.ds(base, CPB)], colv)
        pltpu.sync_copy(row_hbm.at[pl.ds(base, CPB)], rowv)
        for b in range(NBUF):
            pltpu.make_async_copy(g_hbm.at[colv.at[b]], bufs[b], sems[b]).start()

        def inner(g_i, carry2):
            for b in range(NBUF):
                j = NBUF * g_i + b
                pltpu.make_async_copy(g_hbm.at[colv.at[j]], bufs[b], sems[b]).wait()
                pltpu.async_copy(bufs[b], acc.at[rowv.at[j]], ssems[b],
                                 add=True)
                jn = j + NBUF

                @pl.when(jn < CPB)
                def _():
                    pltpu.make_async_copy(
                        bufs[b], acc.at[rowv.at[j]], ssems[b]).wait()
                    pltpu.make_async_copy(
                        g_hbm.at[colv.at[jn]], bufs[b], sems[b]).start()
            return carry2

        lax.fori_loop(0, CPB // NBUF, inner, 0)
        for b in range(NBUF):
            j = CPB - NBUF + b
            pltpu.make_async_copy(bufs[b], acc.at[rowv.at[j]], ssems[b]).wait()
        return carry

    lax.fori_loop(0, NBLK, outer, 0)
    plsc.subcore_barrier()
    pltpu.sync_copy(acc.at[pl.ds(s * ZPT, ZPT)],
                    out_hbm.at[pl.ds(c * N_PAD + s * ZPT, ZPT)])


BR = 400
GRID = N // BR


def _tc_pre_body(x_ref, dp0_ref, dp1_ref, g0_ref, d2_ref, sqd_ref):
    deg = dp0_ref[...] + dp1_ref[...] + 1.0
    dinv = lax.rsqrt(deg)
    g0_ref[...] = x_ref[...] * dinv
    d2_ref[...] = 1.0 / deg
    sqd_ref[...] = deg * dinv


def _tc_precompute(x, dp0, dp1):
    return pl.pallas_call(
        _tc_pre_body,
        grid=(GRID,),
        in_specs=[
            pl.BlockSpec((BR, D), lambda i: (i, 0)),
            pl.BlockSpec((BR, 1), lambda i: (i, 0)),
            pl.BlockSpec((BR, 1), lambda i: (i, 0)),
        ],
        out_specs=[
            pl.BlockSpec((BR, D), lambda i: (i, 0)),
            pl.BlockSpec((BR, 1), lambda i: (i, 0)),
            pl.BlockSpec((BR, 1), lambda i: (i, 0)),
        ],
        out_shape=[
            jax.ShapeDtypeStruct((N, D), jnp.float32),
            jax.ShapeDtypeStruct((N, 1), jnp.float32),
            jax.ShapeDtypeStruct((N, 1), jnp.float32),
        ],
    )(x, dp0, dp1)


def _tc_update_body(g_ref, s0_ref, s1_ref, d2_ref, out_ref):
    g = g_ref[...]
    stot = s0_ref[...] + s1_ref[...] + g
    out_ref[...] = 0.5 * g - 0.5 * d2_ref[...] * stot


def _tc_update(g, s0, s1, d2):
    return pl.pallas_call(
        _tc_update_body,
        grid=(GRID,),
        in_specs=[
            pl.BlockSpec((BR, D), lambda i: (i, 0)),
            pl.BlockSpec((BR, D), lambda i: (i, 0)),
            pl.BlockSpec((BR, D), lambda i: (i, 0)),
            pl.BlockSpec((BR, 1), lambda i: (i, 0)),
        ],
        out_specs=pl.BlockSpec((BR, D), lambda i: (i, 0)),
        out_shape=jax.ShapeDtypeStruct((N, D), jnp.float32),
    )(g, s0, s1, d2)


CMAT = [[0.0] * (K + 1) for _ in range(K + 1)]
for k in range(K + 1):
    for j in range(K - k + 1):
        CMAT[k][k + j] = float(((-1) ** j) * comb(K, k) * comb(K - k, j))


def _tc_combine_body(sqd_ref, *refs):
    g_refs = refs[:K + 1]
    out_ref = refs[K + 1]
    sq = sqd_ref[...]
    gs = [r[...] for r in g_refs]
    for k in range(K + 1):
        acc = None
        for m in range(k, K + 1):
            term = CMAT[k][m] * gs[m]
            acc = term if acc is None else acc + term
        out_ref[k, :, :] = acc * sq


def _tc_combine(sqd, gs):
    in_specs = [pl.BlockSpec((BR, 1), lambda i: (i, 0))]
    in_specs += [pl.BlockSpec((BR, D), lambda i: (i, 0)) for _ in range(K + 1)]
    return pl.pallas_call(
        _tc_combine_body,
        grid=(GRID,),
        in_specs=in_specs,
        out_specs=pl.BlockSpec((K + 1, BR, D), lambda i: (0, i, 0)),
        out_shape=jax.ShapeDtypeStruct((K + 1, N, D), jnp.float32),
    )(sqd, *gs)


@jax.jit
def kernel(x, edge_index):
    row = edge_index[0].astype(jnp.int32)
    col = edge_index[1].astype(jnp.int32)
    pad = E_PAD - E
    row_p = jnp.concatenate([row, jnp.full((pad,), N, jnp.int32)])
    col_p = jnp.concatenate([col, jnp.zeros((pad,), jnp.int32)])
    row2 = row_p.reshape(NW * TCH, CL)
    col2 = col_p.reshape(NW * TCH, CL)

    zeros_d = jnp.zeros((N_PAD, D), jnp.float32)
    ones_nd = jnp.ones((N, D), jnp.float32)

    dsp = _sc_spmm(ones_nd, col2, row2, zeros_d)
    g0, d2, sqd = _tc_precompute(x, dsp[:N, 0:1], dsp[N_PAD:N_PAD + N, 0:1])

    gs = [g0]
    g = g0
    for _ in range(K):
        sparts = _sc_spmm(g, col2, row2, zeros_d)
        g = _tc_update(g, sparts[:N], sparts[N_PAD:N_PAD + N], d2)
        gs.append(g)

    return _tc_combine(sqd, gs)
